# Initial kernel scaffold; baseline (speedup 1.0000x reference)
import jax
import jax.numpy as jnp
from jax.experimental import pallas as pl


def kernel(x, table, W, b):
    raise NotImplementedError("write your pallas kernel here")


# speedup vs baseline: 2.8832x; 2.8832x over previous
import functools

import jax
import jax.numpy as jnp
from jax import lax
from jax.experimental import pallas as pl
from jax.experimental.pallas import tpu as pltpu
from jax.experimental.pallas import tpu_sc as plsc

B = 16384
S = 200
D = 64
HALF = S // 2
NW = 32
ROWS_PER_W = B // NW
CH = 64
NCHUNK = ROWS_PER_W // CH


def _pool_body(x_hbm, table_hbm, out_hbm, idx_v, gat_v, out_v,
               sem_idx, sem_g0, sem_g1, sem_o0, sem_o1):
    wid = lax.axis_index("s") * 2 + lax.axis_index("c")
    base = wid * ROWS_PER_W
    sem_g = (sem_g0, sem_g1)
    sem_o = (sem_o0, sem_o1)

    pltpu.async_copy(x_hbm.at[pl.ds(base, CH)], idx_v.at[0], sem_idx)

    def issue_row(c2, r, bslot):
        for h in (0, 1):
            pltpu.async_copy(table_hbm.at[idx_v.at[c2, r, h]],
                             gat_v.at[bslot, h], sem_g[bslot])

    def wait_row(c2, r, bslot):
        for h in (0, 1):
            pltpu.make_async_copy(table_hbm.at[idx_v.at[c2, r, h]],
                                  gat_v.at[bslot, h], sem_g[bslot]).wait()

    def reduce_row(c2, r, bslot):
        def rbody(s, acc):
            nxt = []
            for h in (0, 1):
                for c in range(4):
                    nxt.append(acc[h * 4 + c]
                               + gat_v[bslot, h, s, pl.ds(c * 16, 16)])
            return tuple(nxt)

        acc = lax.fori_loop(
            0, HALF, rbody,
            tuple(jnp.zeros((16,), jnp.float32) for _ in range(8)))
        for c in range(4):
            out_v[c2, r, pl.ds(c * 16, 16)] = acc[c] + acc[4 + c]

    def process_chunk(q, c2):
        row0 = base + q * CH
        pltpu.make_async_copy(x_hbm.at[pl.ds(row0, CH)], idx_v.at[c2],
                              sem_idx).wait()

        @pl.when(q < NCHUNK - 1)
        def _():
            pltpu.async_copy(x_hbm.at[pl.ds(row0 + CH, CH)],
                             idx_v.at[1 - c2], sem_idx)

        @pl.when(q >= 2)
        def _():
            pltpu.make_async_copy(out_v.at[c2],
                                  out_hbm.at[pl.ds(row0 - 2 * CH, CH)],
                                  sem_o[c2]).wait()

        issue_row(c2, 0, 0)
        issue_row(c2, 1, 1)

        def jbody(j, carry):
            for bslot in (0, 1):
                r = 2 * j + bslot
                wait_row(c2, r, bslot)
                reduce_row(c2, r, bslot)
                issue_row(c2, r + 2, bslot)
            return carry

        lax.fori_loop(0, CH // 2 - 1, jbody, 0)
        for bslot in (0, 1):
            r = CH - 2 + bslot
            wait_row(c2, r, bslot)
            reduce_row(c2, r, bslot)

        pltpu.async_copy(out_v.at[c2], out_hbm.at[pl.ds(row0, CH)],
                         sem_o[c2])

    def qbody(i, carry):
        process_chunk(2 * i, 0)
        process_chunk(2 * i + 1, 1)
        return carry

    lax.fori_loop(0, NCHUNK // 2, qbody, 0)

    for c2 in (0, 1):
        q = NCHUNK - 2 + c2
        row0 = base + q * CH
        pltpu.make_async_copy(out_v.at[c2],
                              out_hbm.at[pl.ds(row0, CH)], sem_o[c2]).wait()


_pool_call = functools.partial(
    pl.kernel,
    out_type=jax.ShapeDtypeStruct((B, D), jnp.float32),
    mesh=plsc.VectorSubcoreMesh(core_axis_name="c", subcore_axis_name="s"),
    compiler_params=pltpu.CompilerParams(use_tc_tiling_on_sc=False),
    scratch_types=[
        pltpu.VMEM((2, CH, 2, HALF), jnp.int32),
        pltpu.VMEM((2, 2, HALF, D), jnp.float32),
        pltpu.VMEM((2, CH, D), jnp.float32),
        pltpu.SemaphoreType.DMA,
        pltpu.SemaphoreType.DMA,
        pltpu.SemaphoreType.DMA,
        pltpu.SemaphoreType.DMA,
        pltpu.SemaphoreType.DMA,
    ],
)(_pool_body)


def _fc_body(p_ref, wt_ref, b_ref, o_ref):
    o_ref[...] = jnp.dot(jnp.maximum(p_ref[...], 0.0), wt_ref[...],
                         preferred_element_type=jnp.float32) + b_ref[...]


def kernel(x, table, W, b):
    x3 = x.reshape(B, 2, HALF)
    pooled = _pool_call(x3, table)
    wt = W.T * (1.0 / S)
    b2 = b.reshape(1, -1)
    nc = W.shape[0]
    blk = 1024
    return pl.pallas_call(
        _fc_body,
        grid=(B // blk,),
        in_specs=[pl.BlockSpec((blk, D), lambda i: (i, 0)),
                  pl.BlockSpec((D, nc), lambda i: (0, 0)),
                  pl.BlockSpec((1, nc), lambda i: (0, 0))],
        out_specs=pl.BlockSpec((blk, nc), lambda i: (i, 0)),
        out_shape=jax.ShapeDtypeStruct((B, nc), jnp.float32),
    )(pooled, wt, b2)


# speedup vs baseline: 3.5513x; 1.2317x over previous
import functools

import jax
import jax.numpy as jnp
from jax import lax
from jax.experimental import pallas as pl
from jax.experimental.pallas import tpu as pltpu
from jax.experimental.pallas import tpu_sc as plsc

B = 16384
S = 200
D = 64
HALF = S // 2
NW = 32
ROWS_PER_W = B // NW
CH = 64
NCHUNK = ROWS_PER_W // CH


def _pool_body(x_flat_hbm, table_flat_hbm, out_hbm, idx_v, gat_v, out_v,
               sem_idx, sem_g0, sem_g1, sem_o0, sem_o1):
    x_hbm = x_flat_hbm
    table_hbm = table_flat_hbm
    wid = lax.axis_index("s") * 2 + lax.axis_index("c")
    base = wid * ROWS_PER_W
    sem_g = (sem_g0, sem_g1)
    sem_o = (sem_o0, sem_o1)

    pltpu.async_copy(x_hbm.at[pl.ds(base, CH)], idx_v.at[0], sem_idx)

    def issue_row(c2, r, bslot):
        for h in (0, 1):
            pltpu.async_copy(table_hbm.at[idx_v.at[c2, r, h]],
                             gat_v.at[bslot, h], sem_g[bslot])

    def wait_row(c2, r, bslot):
        for h in (0, 1):
            pltpu.make_async_copy(table_hbm.at[idx_v.at[c2, r, h]],
                                  gat_v.at[bslot, h], sem_g[bslot]).wait()

    def reduce_row(c2, r, bslot):
        def rbody(s, acc):
            nxt = []
            for h in (0, 1):
                for c in range(4):
                    nxt.append(acc[h * 4 + c]
                               + gat_v[bslot, h, s, pl.ds(c * 16, 16)])
            return tuple(nxt)

        acc = lax.fori_loop(
            0, HALF, rbody,
            tuple(jnp.zeros((16,), jnp.float32) for _ in range(8)))
        for c in range(4):
            out_v[c2, r, pl.ds(c * 16, 16)] = acc[c] + acc[4 + c]

    def process_chunk(q, c2):
        row0 = base + q * CH
        pltpu.make_async_copy(x_hbm.at[pl.ds(row0, CH)], idx_v.at[c2],
                              sem_idx).wait()

        @pl.when(q < NCHUNK - 1)
        def _():
            pltpu.async_copy(x_hbm.at[pl.ds(row0 + CH, CH)],
                             idx_v.at[1 - c2], sem_idx)

        @pl.when(q >= 2)
        def _():
            pltpu.make_async_copy(out_v.at[c2],
                                  out_hbm.at[pl.ds(row0 - 2 * CH, CH)],
                                  sem_o[c2]).wait()

        issue_row(c2, 0, 0)
        issue_row(c2, 1, 1)

        def jbody(j, carry):
            for bslot in (0, 1):
                r = 2 * j + bslot
                wait_row(c2, r, bslot)
                reduce_row(c2, r, bslot)
                issue_row(c2, r + 2, bslot)
            return carry

        lax.fori_loop(0, CH // 2 - 1, jbody, 0)
        for bslot in (0, 1):
            r = CH - 2 + bslot
            wait_row(c2, r, bslot)
            reduce_row(c2, r, bslot)

        pltpu.async_copy(out_v.at[c2], out_hbm.at[pl.ds(row0, CH)],
                         sem_o[c2])

    def qbody(i, carry):
        process_chunk(2 * i, 0)
        process_chunk(2 * i + 1, 1)
        return carry

    lax.fori_loop(0, NCHUNK // 2, qbody, 0)

    for c2 in (0, 1):
        q = NCHUNK - 2 + c2
        row0 = base + q * CH
        pltpu.make_async_copy(out_v.at[c2],
                              out_hbm.at[pl.ds(row0, CH)], sem_o[c2]).wait()


_pool_call = functools.partial(
    pl.kernel,
    out_type=jax.ShapeDtypeStruct((B, D), jnp.float32),
    mesh=plsc.VectorSubcoreMesh(core_axis_name="c", subcore_axis_name="s"),
    compiler_params=pltpu.CompilerParams(use_tc_tiling_on_sc=False),
    scratch_types=[
        pltpu.VMEM((2, CH, 2, HALF), jnp.int32),
        pltpu.VMEM((2, 2, HALF, D), jnp.float32),
        pltpu.VMEM((2, CH, D), jnp.float32),
        pltpu.SemaphoreType.DMA,
        pltpu.SemaphoreType.DMA,
        pltpu.SemaphoreType.DMA,
        pltpu.SemaphoreType.DMA,
        pltpu.SemaphoreType.DMA,
    ],
)(_pool_body)


TN = 2048


def _tr_body(tin_ref, tout_ref):
    blk = tin_ref[...]
    tout_ref[:, 0:D] = blk[:, 0:TN].T
    tout_ref[:, D:2 * D] = blk[:, TN:2 * TN].T


def _fc_body(p_ref, wt_ref, b_ref, o_ref):
    o_ref[...] = jnp.dot(jnp.maximum(p_ref[...], 0.0), wt_ref[...],
                         preferred_element_type=jnp.float32) + b_ref[...]


def kernel(x, table, W, b):
    vocab = table.shape[0]
    nblk = (vocab + 2 * TN - 1) // (2 * TN)
    t2 = pl.pallas_call(
        _tr_body,
        grid=(nblk,),
        in_specs=[pl.BlockSpec((D, 2 * TN), lambda i: (0, i))],
        out_specs=pl.BlockSpec((TN, 128), lambda i: (i, 0)),
        out_shape=jax.ShapeDtypeStruct((nblk * TN, 128), jnp.float32),
    )(table.T)
    t_lin = t2.reshape(nblk * TN * 2, D)
    xr = (x & ~(2 * TN - 1)) | ((x & (TN - 1)) << 1) | ((x >> 11) & 1)
    pooled = _pool_call(xr.reshape(B, 2, HALF), t_lin)
    wt = W.T * (1.0 / S)
    b2 = b.reshape(1, -1)
    nc = W.shape[0]
    blk = 1024
    return pl.pallas_call(
        _fc_body,
        grid=(B // blk,),
        in_specs=[pl.BlockSpec((blk, D), lambda i: (i, 0)),
                  pl.BlockSpec((D, nc), lambda i: (0, 0)),
                  pl.BlockSpec((1, nc), lambda i: (0, 0))],
        out_specs=pl.BlockSpec((blk, nc), lambda i: (i, 0)),
        out_shape=jax.ShapeDtypeStruct((B, nc), jnp.float32),
    )(pooled, wt, b2)


# speedup vs baseline: 3.8118x; 1.0734x over previous
import functools

import jax
import jax.numpy as jnp
from jax import lax
from jax.experimental import pallas as pl
from jax.experimental.pallas import tpu as pltpu
from jax.experimental.pallas import tpu_sc as plsc

B = 16384
S = 200
D = 64
HALF = S // 2
NW = 32
ROWS_PER_W = B // NW
CH = 64
NCHUNK = ROWS_PER_W // CH


def _pool_body(x_flat_hbm, table_flat_hbm, out_hbm, idx_v, gat_v, out_v,
               sem_idx, sem_g0, sem_g1, sem_o0, sem_o1):
    x_hbm = x_flat_hbm
    table_hbm = table_flat_hbm
    wid = lax.axis_index("s") * 2 + lax.axis_index("c")
    base = wid * ROWS_PER_W
    sem_g = (sem_g0, sem_g1)
    sem_o = (sem_o0, sem_o1)

    pltpu.async_copy(x_hbm.at[pl.ds(base, CH)], idx_v.at[0], sem_idx)

    def issue_row(c2, r, bslot):
        pltpu.async_copy(table_hbm.at[idx_v.at[c2, r]],
                         gat_v.at[bslot], sem_g[bslot])

    def wait_row(c2, r, bslot):
        pltpu.make_async_copy(table_hbm.at[idx_v.at[c2, r]],
                              gat_v.at[bslot], sem_g[bslot]).wait()

    def reduce_row(c2, r, bslot):
        def rbody(s, acc):
            nxt = []
            for h in (0, 1):
                for c in range(4):
                    nxt.append(acc[h * 4 + c]
                               + gat_v[bslot, HALF * h + s, pl.ds(c * 16, 16)])
            return tuple(nxt)

        acc = lax.fori_loop(
            0, HALF, rbody,
            tuple(jnp.zeros((16,), jnp.float32) for _ in range(8)))
        for c in range(4):
            out_v[c2, r, pl.ds(c * 16, 16)] = acc[c] + acc[4 + c]

    def process_chunk(q, c2):
        row0 = base + q * CH
        pltpu.make_async_copy(x_hbm.at[pl.ds(row0, CH)], idx_v.at[c2],
                              sem_idx).wait()

        @pl.when(q < NCHUNK - 1)
        def _():
            pltpu.async_copy(x_hbm.at[pl.ds(row0 + CH, CH)],
                             idx_v.at[1 - c2], sem_idx)

        @pl.when(q >= 2)
        def _():
            pltpu.make_async_copy(out_v.at[c2],
                                  out_hbm.at[pl.ds(row0 - 2 * CH, CH)],
                                  sem_o[c2]).wait()

        issue_row(c2, 0, 0)
        issue_row(c2, 1, 1)

        def jbody(j, carry):
            for bslot in (0, 1):
                r = 2 * j + bslot
                wait_row(c2, r, bslot)
                reduce_row(c2, r, bslot)
                issue_row(c2, r + 2, bslot)
            return carry

        lax.fori_loop(0, CH // 2 - 1, jbody, 0)
        for bslot in (0, 1):
            r = CH - 2 + bslot
            wait_row(c2, r, bslot)
            reduce_row(c2, r, bslot)

        pltpu.async_copy(out_v.at[c2], out_hbm.at[pl.ds(row0, CH)],
                         sem_o[c2])

    def qbody(i, carry):
        process_chunk(2 * i, 0)
        process_chunk(2 * i + 1, 1)
        return carry

    lax.fori_loop(0, NCHUNK // 2, qbody, 0)

    for c2 in (0, 1):
        q = NCHUNK - 2 + c2
        row0 = base + q * CH
        pltpu.make_async_copy(out_v.at[c2],
                              out_hbm.at[pl.ds(row0, CH)], sem_o[c2]).wait()


_pool_call = functools.partial(
    pl.kernel,
    out_type=jax.ShapeDtypeStruct((B, D), jnp.float32),
    mesh=plsc.VectorSubcoreMesh(core_axis_name="c", subcore_axis_name="s"),
    compiler_params=pltpu.CompilerParams(use_tc_tiling_on_sc=False),
    scratch_types=[
        pltpu.VMEM((2, CH, S), jnp.int32),
        pltpu.VMEM((2, S, D), jnp.float32),
        pltpu.VMEM((2, CH, D), jnp.float32),
        pltpu.SemaphoreType.DMA,
        pltpu.SemaphoreType.DMA,
        pltpu.SemaphoreType.DMA,
        pltpu.SemaphoreType.DMA,
        pltpu.SemaphoreType.DMA,
    ],
)(_pool_body)


TN = 2048


def _tr_body(tin_ref, tout_ref):
    blk = tin_ref[...]
    tout_ref[:, 0:D] = blk[:, 0:TN].T
    tout_ref[:, D:2 * D] = blk[:, TN:2 * TN].T


def _fc_body(p_ref, wt_ref, b_ref, o_ref):
    o_ref[...] = jnp.dot(jnp.maximum(p_ref[...], 0.0), wt_ref[...],
                         preferred_element_type=jnp.float32) + b_ref[...]


def kernel(x, table, W, b):
    vocab = table.shape[0]
    nblk = (vocab + 2 * TN - 1) // (2 * TN)
    t2 = pl.pallas_call(
        _tr_body,
        grid=(nblk,),
        in_specs=[pl.BlockSpec((D, 2 * TN), lambda i: (0, i))],
        out_specs=pl.BlockSpec((TN, 128), lambda i: (i, 0)),
        out_shape=jax.ShapeDtypeStruct((nblk * TN, 128), jnp.float32),
    )(table.T)
    t_lin = t2.reshape(nblk * TN * 2, D)
    xr = (x & ~(2 * TN - 1)) | ((x & (TN - 1)) << 1) | ((x >> 11) & 1)
    pooled = _pool_call(xr, t_lin)
    wt = W.T * (1.0 / S)
    b2 = b.reshape(1, -1)
    nc = W.shape[0]
    blk = 1024
    return pl.pallas_call(
        _fc_body,
        grid=(B // blk,),
        in_specs=[pl.BlockSpec((blk, D), lambda i: (i, 0)),
                  pl.BlockSpec((D, nc), lambda i: (0, 0)),
                  pl.BlockSpec((1, nc), lambda i: (0, 0))],
        out_specs=pl.BlockSpec((blk, nc), lambda i: (i, 0)),
        out_shape=jax.ShapeDtypeStruct((B, nc), jnp.float32),
    )(pooled, wt, b2)


# speedup vs baseline: 4.2151x; 1.1058x over previous
import functools

import jax
import jax.numpy as jnp
from jax import lax
from jax.experimental import pallas as pl
from jax.experimental.pallas import tpu as pltpu
from jax.experimental.pallas import tpu_sc as plsc

B = 16384
S = 200
D = 64
HALF = S // 2
NW = 32
ROWS_PER_W = B // NW
G = 2
GS = G * S
CH = 32
NCHUNK = ROWS_PER_W // (G * CH)


def _pool_body(x_flat_hbm, table_flat_hbm, out_hbm, idx_v, gat_v, out_v,
               sem_idx, sem_g0, sem_g1, sem_o0, sem_o1):
    x_hbm = x_flat_hbm
    table_hbm = table_flat_hbm
    wid = lax.axis_index("s") * 2 + lax.axis_index("c")
    base = wid * (ROWS_PER_W // G)
    sem_g = (sem_g0, sem_g1)
    sem_o = (sem_o0, sem_o1)

    pltpu.async_copy(x_hbm.at[pl.ds(base, CH)], idx_v.at[0], sem_idx)

    def issue_row(c2, r, bslot):
        pltpu.async_copy(table_hbm.at[idx_v.at[c2, r]],
                         gat_v.at[bslot], sem_g[bslot])

    def wait_row(c2, r, bslot):
        pltpu.make_async_copy(table_hbm.at[idx_v.at[c2, r]],
                              gat_v.at[bslot], sem_g[bslot]).wait()

    def reduce_row(c2, r, bslot):
        def rbody(s, acc):
            nxt = []
            for g in range(G):
                for h in (0, 1):
                    for c in range(4):
                        nxt.append(acc[(g * 2 + h) * 4 + c]
                                   + gat_v[bslot, g * S + HALF * h + s,
                                           pl.ds(c * 16, 16)])
            return tuple(nxt)

        acc = lax.fori_loop(
            0, HALF, rbody,
            tuple(jnp.zeros((16,), jnp.float32) for _ in range(8 * G)))
        for g in range(G):
            for c in range(4):
                out_v[c2, G * r + g, pl.ds(c * 16, 16)] = (
                    acc[(g * 2) * 4 + c] + acc[(g * 2 + 1) * 4 + c])

    def process_chunk(q, c2):
        row0 = base + q * CH
        pltpu.make_async_copy(x_hbm.at[pl.ds(row0, CH)], idx_v.at[c2],
                              sem_idx).wait()

        @pl.when(q < NCHUNK - 1)
        def _():
            pltpu.async_copy(x_hbm.at[pl.ds(row0 + CH, CH)],
                             idx_v.at[1 - c2], sem_idx)

        @pl.when(q >= 2)
        def _():
            pltpu.make_async_copy(out_v.at[c2],
                                  out_hbm.at[pl.ds((row0 - 2 * CH) * G,
                                                   G * CH)],
                                  sem_o[c2]).wait()

        issue_row(c2, 0, 0)
        issue_row(c2, 1, 1)

        def jbody(j, carry):
            for bslot in (0, 1):
                r = 2 * j + bslot
                wait_row(c2, r, bslot)
                reduce_row(c2, r, bslot)
                issue_row(c2, r + 2, bslot)
            return carry

        lax.fori_loop(0, CH // 2 - 1, jbody, 0)
        for bslot in (0, 1):
            r = CH - 2 + bslot
            wait_row(c2, r, bslot)
            reduce_row(c2, r, bslot)

        pltpu.async_copy(out_v.at[c2], out_hbm.at[pl.ds(row0 * G, G * CH)],
                         sem_o[c2])

    def qbody(i, carry):
        process_chunk(2 * i, 0)
        process_chunk(2 * i + 1, 1)
        return carry

    lax.fori_loop(0, NCHUNK // 2, qbody, 0)

    for c2 in (0, 1):
        q = NCHUNK - 2 + c2
        row0 = base + q * CH
        pltpu.make_async_copy(out_v.at[c2],
                              out_hbm.at[pl.ds(row0 * G, G * CH)],
                              sem_o[c2]).wait()


_pool_call = functools.partial(
    pl.kernel,
    out_type=jax.ShapeDtypeStruct((B, D), jnp.float32),
    mesh=plsc.VectorSubcoreMesh(core_axis_name="c", subcore_axis_name="s"),
    compiler_params=pltpu.CompilerParams(use_tc_tiling_on_sc=False),
    scratch_types=[
        pltpu.VMEM((2, CH, GS), jnp.int32),
        pltpu.VMEM((2, GS, D), jnp.float32),
        pltpu.VMEM((2, G * CH, D), jnp.float32),
        pltpu.SemaphoreType.DMA,
        pltpu.SemaphoreType.DMA,
        pltpu.SemaphoreType.DMA,
        pltpu.SemaphoreType.DMA,
        pltpu.SemaphoreType.DMA,
    ],
)(_pool_body)


TN = 2048


def _tr_body(tin_ref, tout_ref):
    blk = tin_ref[...]
    tout_ref[:, 0:D] = blk[:, 0:TN].T
    tout_ref[:, D:2 * D] = blk[:, TN:2 * TN].T


def _fc_body(p_ref, wt_ref, b_ref, o_ref):
    o_ref[...] = jnp.dot(jnp.maximum(p_ref[...], 0.0), wt_ref[...],
                         preferred_element_type=jnp.float32) + b_ref[...]


def kernel(x, table, W, b):
    vocab = table.shape[0]
    nblk = (vocab + 2 * TN - 1) // (2 * TN)
    t2 = pl.pallas_call(
        _tr_body,
        grid=(nblk,),
        in_specs=[pl.BlockSpec((D, 2 * TN), lambda i: (0, i))],
        out_specs=pl.BlockSpec((TN, 128), lambda i: (i, 0)),
        out_shape=jax.ShapeDtypeStruct((nblk * TN, 128), jnp.float32),
    )(table.T)
    t_lin = t2.reshape(nblk * TN * 2, D)
    xr = (x & ~(2 * TN - 1)) | ((x & (TN - 1)) << 1) | ((x >> 11) & 1)
    pooled = _pool_call(xr.reshape(B // G, GS), t_lin)
    wt = W.T * (1.0 / S)
    b2 = b.reshape(1, -1)
    nc = W.shape[0]
    blk = 1024
    return pl.pallas_call(
        _fc_body,
        grid=(B // blk,),
        in_specs=[pl.BlockSpec((blk, D), lambda i: (i, 0)),
                  pl.BlockSpec((D, nc), lambda i: (0, 0)),
                  pl.BlockSpec((1, nc), lambda i: (0, 0))],
        out_specs=pl.BlockSpec((blk, nc), lambda i: (i, 0)),
        out_shape=jax.ShapeDtypeStruct((B, nc), jnp.float32),
    )(pooled, wt, b2)
